# Initial kernel scaffold; baseline (speedup 1.0000x reference)
import jax
import jax.numpy as jnp
from jax.experimental import pallas as pl


def kernel(x_embed, prompt):
    raise NotImplementedError("write your pallas kernel here")


# speedup vs baseline: 5.3646x; 5.3646x over previous
import functools

import jax
import jax.numpy as jnp
from jax import lax
from jax.experimental import pallas as pl
from jax.experimental.pallas import tpu as pltpu
from jax.experimental.pallas import tpu_sc as plsc

POOL = 8192
LEN = 5
DIM = 64
TOPK = 8
BATCH = 4096

BBLK = 256
NBLK = BATCH // BBLK

PADW = 384
PBLK = 512
NPB = POOL // PBLK

N_ROWS = BATCH * TOPK * LEN
RBLK = 2048
NRB = (BATCH * TOPK) // RBLK

_F32 = jnp.float32


def _rowsq(v):
    a = None
    for c in range(8):
        s = v[:, 8 * c:8 * c + 8]
        s = s * s
        a = s if a is None else s + a
    b = a[:, 4:8] + a[:, 0:4]
    c2 = b[:, 2:4] + b[:, 0:2]
    return c2[:, 1:2] + c2[:, 0:1]


def _prompt_norm_body(p_ref, pn_ref, tab_ref):
    p = p_ref[...]
    slices = [p[:, l, :] for l in range(LEN)]
    zpad = jnp.zeros((PBLK, PADW - LEN * DIM), _F32)
    tab_ref[...] = jnp.concatenate(slices + [zpad], axis=1)
    pk = sum(slices[1:], slices[0]) / _F32(LEN)
    sq = _rowsq(pk)
    pn_ref[...] = pk * lax.rsqrt(jnp.maximum(sq, _F32(1e-12)))


def _sim_topk_body(x_ref, pn_ref, sim_ref, idx_ref, acc_ref):
    i = pl.program_id(0)
    x = x_ref[...]
    sq = _rowsq(x)
    xn = x * lax.rsqrt(jnp.maximum(sq, _F32(1e-12)))
    pn = pn_ref[...]
    sim = lax.dot_general(xn, pn, (((1,), (1,)), ((), ())),
                          preferred_element_type=_F32)
    sim_ref[...] = sim
    iota = lax.broadcasted_iota(jnp.int32, (BBLK, POOL), 1)
    col = lax.broadcasted_iota(jnp.int32, (BBLK, TOPK), 1)
    masked = sim
    idx_acc = jnp.zeros((BBLK, TOPK), jnp.int32)
    tot = _F32(0.0)
    neg = _F32(float("-inf"))
    for k in range(TOPK):
        m = jnp.max(masked, axis=1, keepdims=True)
        cand = jnp.where(masked == m, iota, POOL)
        arg = jnp.min(cand, axis=1, keepdims=True)
        idx_acc = jnp.where(col == k, arg, idx_acc)
        tot = tot + jnp.sum(m)
        masked = jnp.where(iota == arg, neg, masked)
    idx_ref[...] = idx_acc

    @pl.when(i == 0)
    def _():
        acc_ref[...] = jnp.zeros_like(acc_ref)

    acc_ref[...] += tot * _F32(1.0 / BATCH)


def _corr_body(fp_ref, loss_ref, bp_ref, g_acc, s_acc):
    i = pl.program_id(0)
    fp = fp_ref[...]
    bp_ref[...] = fp[:, : LEN * DIM]
    g = jnp.zeros((DIM, DIM), _F32)
    cs = jnp.zeros((1, DIM), _F32)
    for j in range(LEN):
        fj = fp[:, j * DIM:(j + 1) * DIM]
        g = g + lax.dot_general(fj, fj, (((0,), (0,)), ((), ())),
                                preferred_element_type=_F32)
        cs = cs + jnp.sum(fj, axis=0, keepdims=True)

    @pl.when(i == 0)
    def _():
        g_acc[...] = jnp.zeros_like(g_acc)
        s_acc[...] = jnp.zeros_like(s_acc)

    g_acc[...] += g
    s_acc[...] += cs

    @pl.when(i == NRB - 1)
    def _():
        n = _F32(N_ROWS)
        g_full = g_acc[...]
        m = s_acc[...] * _F32(1.0 / N_ROWS)
        outer = lax.dot_general(m, m, (((0,), (0,)), ((), ())),
                                preferred_element_type=_F32)
        cov = g_full - n * outer
        ri = lax.broadcasted_iota(jnp.int32, (DIM, DIM), 0)
        ci = lax.broadcasted_iota(jnp.int32, (DIM, DIM), 1)
        eye = (ri == ci).astype(_F32)
        inv_nm1 = _F32(1.0 / (N_ROWS - 1))
        var_col = jnp.sum(cov * eye, axis=0, keepdims=True) * inv_nm1
        var_row = jnp.sum(cov * eye, axis=1, keepdims=True) * inv_nm1
        eps = _F32(1e-6)
        denom = (jnp.sqrt(var_row) + eps) * (jnp.sqrt(var_col) + eps)
        c = cov * _F32(1.0 / N_ROWS) / denom
        on_diag = jnp.sum((c * eye - eye) ** 2, keepdims=True) * _F32(1.0 / DIM)
        off_diag = jnp.sum((c * (1.0 - eye)) ** 2, keepdims=True) * _F32(
            1.0 / (DIM * DIM - DIM))
        loss_ref[...] = on_diag + _F32(0.005) * off_diag


_NC = 2
_NS = 16
_NW = _NC * _NS
_BPW = (BATCH * TOPK) // _NW
_CHUNK = 128
_NCHUNK = _BPW // _CHUNK


def _sc_gather_body(table_hbm, idx_hbm, out_hbm, idx_v, rows_v, sem):
    wid = lax.axis_index("s") * _NC + lax.axis_index("c")
    base = wid * _BPW
    pltpu.sync_copy(idx_hbm.at[pl.ds(base, _BPW)], idx_v)
    for c in range(_NCHUNK):
        pltpu.async_copy(
            table_hbm.at[idx_v.at[pl.ds(c * _CHUNK, _CHUNK)]],
            rows_v, sem).wait()
        pltpu.sync_copy(rows_v, out_hbm.at[pl.ds(base + c * _CHUNK, _CHUNK)])


@functools.cache
def _sc_gather_fn():
    return pl.kernel(
        _sc_gather_body,
        mesh=plsc.VectorSubcoreMesh(core_axis_name="c", subcore_axis_name="s",
                                    num_cores=_NC, num_subcores=_NS),
        out_type=jax.ShapeDtypeStruct((BATCH * TOPK, PADW), _F32),
        scratch_types=[
            pltpu.VMEM((_BPW,), jnp.int32),
            pltpu.VMEM((_CHUNK, PADW), _F32),
            pltpu.SemaphoreType.DMA,
        ],
    )


def _sc_gather(table, idx_flat):
    return _sc_gather_fn()(table, idx_flat)


def kernel(x_embed, prompt):
    pn, table = pl.pallas_call(
        _prompt_norm_body,
        grid=(NPB,),
        in_specs=[pl.BlockSpec((PBLK, LEN, DIM), lambda i: (i, 0, 0))],
        out_specs=[
            pl.BlockSpec((PBLK, DIM), lambda i: (i, 0)),
            pl.BlockSpec((PBLK, PADW), lambda i: (i, 0)),
        ],
        out_shape=[
            jax.ShapeDtypeStruct((POOL, DIM), _F32),
            jax.ShapeDtypeStruct((POOL, PADW), _F32),
        ],
    )(prompt)

    sim, idx, acc = pl.pallas_call(
        _sim_topk_body,
        grid=(NBLK,),
        in_specs=[
            pl.BlockSpec((BBLK, DIM), lambda i: (i, 0)),
            pl.BlockSpec((POOL, DIM), lambda i: (0, 0)),
        ],
        out_specs=[
            pl.BlockSpec((BBLK, POOL), lambda i: (i, 0)),
            pl.BlockSpec((BBLK, TOPK), lambda i: (i, 0)),
            pl.BlockSpec((1, 1), lambda i: (0, 0)),
        ],
        out_shape=[
            jax.ShapeDtypeStruct((BATCH, POOL), _F32),
            jax.ShapeDtypeStruct((BATCH, TOPK), jnp.int32),
            jax.ShapeDtypeStruct((1, 1), _F32),
        ],
    )(x_embed, pn)

    bp_pad = _sc_gather(table, idx.reshape(BATCH * TOPK))

    loss, bp = pl.pallas_call(
        _corr_body,
        grid=(NRB,),
        in_specs=[pl.BlockSpec((RBLK, PADW), lambda i: (i, 0))],
        out_specs=[
            pl.BlockSpec((1, 1), lambda i: (0, 0)),
            pl.BlockSpec((RBLK, LEN * DIM), lambda i: (i, 0)),
        ],
        out_shape=[
            jax.ShapeDtypeStruct((1, 1), _F32),
            jax.ShapeDtypeStruct((BATCH * TOPK, LEN * DIM), _F32),
        ],
        scratch_shapes=[
            pltpu.VMEM((DIM, DIM), _F32),
            pltpu.VMEM((1, DIM), _F32),
        ],
    )(bp_pad)

    batched_prompt = bp.reshape(BATCH, TOPK * LEN, DIM)
    return (batched_prompt, sim, acc.reshape(()), loss.reshape(()), idx)


# speedup vs baseline: 5.4010x; 1.0068x over previous
import functools

import jax
import jax.numpy as jnp
from jax import lax
from jax.experimental import pallas as pl
from jax.experimental.pallas import tpu as pltpu
from jax.experimental.pallas import tpu_sc as plsc

POOL = 8192
LEN = 5
DIM = 64
TOPK = 8
BATCH = 4096

BBLK = 256
NBLK = BATCH // BBLK

PADW = 384
PBLK = 512
NPB = POOL // PBLK

N_ROWS = BATCH * TOPK * LEN
RBLK = 2048
NRB = (BATCH * TOPK) // RBLK

_F32 = jnp.float32


def _rowsq(v):
    a = None
    for c in range(8):
        s = v[:, 8 * c:8 * c + 8]
        s = s * s
        a = s if a is None else s + a
    b = a[:, 4:8] + a[:, 0:4]
    c2 = b[:, 2:4] + b[:, 0:2]
    return c2[:, 1:2] + c2[:, 0:1]


def _prompt_norm_body(p_ref, pn_ref, tab_ref):
    p = p_ref[...]
    slices = [p[:, l, :] for l in range(LEN)]
    zpad = jnp.zeros((PBLK, PADW - LEN * DIM), _F32)
    tab_ref[...] = jnp.concatenate(slices + [zpad], axis=1)
    pk = sum(slices[1:], slices[0]) / _F32(LEN)
    sq = _rowsq(pk)
    pn_ref[...] = pk * lax.rsqrt(jnp.maximum(sq, _F32(1e-12)))


def _sim_topk_body(x_ref, pn_ref, sim_ref, idx_ref, acc_ref):
    i = pl.program_id(0)
    x = x_ref[...]
    sq = _rowsq(x)
    xn = x * lax.rsqrt(jnp.maximum(sq, _F32(1e-12)))
    pn = pn_ref[...]
    sim = lax.dot_general(xn, pn, (((1,), (1,)), ((), ())),
                          preferred_element_type=_F32)
    sim_ref[...] = sim
    iota = lax.broadcasted_iota(jnp.int32, (BBLK, POOL), 1)
    col = lax.broadcasted_iota(jnp.int32, (BBLK, TOPK), 1)
    masked = sim
    idx_acc = jnp.zeros((BBLK, TOPK), jnp.int32)
    tot = _F32(0.0)
    neg = _F32(float("-inf"))
    for k in range(TOPK):
        m = jnp.max(masked, axis=1, keepdims=True)
        cand = jnp.where(masked == m, iota, POOL)
        arg = jnp.min(cand, axis=1, keepdims=True)
        idx_acc = jnp.where(col == k, arg, idx_acc)
        tot = tot + jnp.sum(m)
        masked = jnp.where(iota == arg, neg, masked)
    idx_ref[...] = idx_acc

    @pl.when(i == 0)
    def _():
        acc_ref[...] = jnp.zeros_like(acc_ref)

    acc_ref[...] += tot * _F32(1.0 / BATCH)


def _corr_body(fp_ref, loss_ref, bp_ref, g_acc, s_acc):
    i = pl.program_id(0)
    fp = fp_ref[...]
    bp_ref[...] = fp[:, : LEN * DIM]
    g = jnp.zeros((DIM, DIM), _F32)
    cs = jnp.zeros((1, DIM), _F32)
    for j in range(LEN):
        fj = fp[:, j * DIM:(j + 1) * DIM]
        g = g + lax.dot_general(fj, fj, (((0,), (0,)), ((), ())),
                                preferred_element_type=_F32)
        cs = cs + jnp.sum(fj, axis=0, keepdims=True)

    @pl.when(i == 0)
    def _():
        g_acc[...] = jnp.zeros_like(g_acc)
        s_acc[...] = jnp.zeros_like(s_acc)

    g_acc[...] += g
    s_acc[...] += cs

    @pl.when(i == NRB - 1)
    def _():
        n = _F32(N_ROWS)
        g_full = g_acc[...]
        m = s_acc[...] * _F32(1.0 / N_ROWS)
        outer = lax.dot_general(m, m, (((0,), (0,)), ((), ())),
                                preferred_element_type=_F32)
        cov = g_full - n * outer
        ri = lax.broadcasted_iota(jnp.int32, (DIM, DIM), 0)
        ci = lax.broadcasted_iota(jnp.int32, (DIM, DIM), 1)
        eye = (ri == ci).astype(_F32)
        inv_nm1 = _F32(1.0 / (N_ROWS - 1))
        var_col = jnp.sum(cov * eye, axis=0, keepdims=True) * inv_nm1
        var_row = jnp.sum(cov * eye, axis=1, keepdims=True) * inv_nm1
        eps = _F32(1e-6)
        denom = (jnp.sqrt(var_row) + eps) * (jnp.sqrt(var_col) + eps)
        c = cov * _F32(1.0 / N_ROWS) / denom
        on_diag = jnp.sum((c * eye - eye) ** 2, keepdims=True) * _F32(1.0 / DIM)
        off_diag = jnp.sum((c * (1.0 - eye)) ** 2, keepdims=True) * _F32(
            1.0 / (DIM * DIM - DIM))
        loss_ref[...] = on_diag + _F32(0.005) * off_diag


_NC = 2
_NS = 16
_NW = _NC * _NS
_BPW = (BATCH * TOPK) // _NW
_CHUNK = 128
_NCHUNK = _BPW // _CHUNK


def _sc_gather_body(table_hbm, idx_hbm, out_hbm, idx_v, rows_v, sem0, sem1):
    wid = lax.axis_index("s") * _NC + lax.axis_index("c")
    base = wid * _BPW
    pltpu.sync_copy(idx_hbm.at[pl.ds(base, _BPW)], idx_v)
    sems = (sem0, sem1)
    cps = [None, None]
    for c in range(_NCHUNK):
        cps[c % 2] = pltpu.async_copy(
            table_hbm.at[idx_v.at[pl.ds(c * _CHUNK, _CHUNK)]],
            rows_v.at[c % 2], sems[c % 2])
        if c > 0:
            cps[(c - 1) % 2].wait()
            pltpu.sync_copy(rows_v.at[(c - 1) % 2],
                            out_hbm.at[pl.ds(base + (c - 1) * _CHUNK, _CHUNK)])
    last = _NCHUNK - 1
    cps[last % 2].wait()
    pltpu.sync_copy(rows_v.at[last % 2],
                    out_hbm.at[pl.ds(base + last * _CHUNK, _CHUNK)])


@functools.cache
def _sc_gather_fn():
    return pl.kernel(
        _sc_gather_body,
        mesh=plsc.VectorSubcoreMesh(core_axis_name="c", subcore_axis_name="s",
                                    num_cores=_NC, num_subcores=_NS),
        out_type=jax.ShapeDtypeStruct((BATCH * TOPK, PADW), _F32),
        scratch_types=[
            pltpu.VMEM((_BPW,), jnp.int32),
            pltpu.VMEM((2, _CHUNK, PADW), _F32),
            pltpu.SemaphoreType.DMA,
            pltpu.SemaphoreType.DMA,
        ],
    )


def _sc_gather(table, idx_flat):
    return _sc_gather_fn()(table, idx_flat)


def kernel(x_embed, prompt):
    pn, table = pl.pallas_call(
        _prompt_norm_body,
        grid=(NPB,),
        in_specs=[pl.BlockSpec((PBLK, LEN, DIM), lambda i: (i, 0, 0))],
        out_specs=[
            pl.BlockSpec((PBLK, DIM), lambda i: (i, 0)),
            pl.BlockSpec((PBLK, PADW), lambda i: (i, 0)),
        ],
        out_shape=[
            jax.ShapeDtypeStruct((POOL, DIM), _F32),
            jax.ShapeDtypeStruct((POOL, PADW), _F32),
        ],
    )(prompt)

    sim, idx, acc = pl.pallas_call(
        _sim_topk_body,
        grid=(NBLK,),
        in_specs=[
            pl.BlockSpec((BBLK, DIM), lambda i: (i, 0)),
            pl.BlockSpec((POOL, DIM), lambda i: (0, 0)),
        ],
        out_specs=[
            pl.BlockSpec((BBLK, POOL), lambda i: (i, 0)),
            pl.BlockSpec((BBLK, TOPK), lambda i: (i, 0)),
            pl.BlockSpec((1, 1), lambda i: (0, 0)),
        ],
        out_shape=[
            jax.ShapeDtypeStruct((BATCH, POOL), _F32),
            jax.ShapeDtypeStruct((BATCH, TOPK), jnp.int32),
            jax.ShapeDtypeStruct((1, 1), _F32),
        ],
    )(x_embed, pn)

    bp_pad = _sc_gather(table, idx.reshape(BATCH * TOPK))

    loss, bp = pl.pallas_call(
        _corr_body,
        grid=(NRB,),
        in_specs=[pl.BlockSpec((RBLK, PADW), lambda i: (i, 0))],
        out_specs=[
            pl.BlockSpec((1, 1), lambda i: (0, 0)),
            pl.BlockSpec((RBLK, LEN * DIM), lambda i: (i, 0)),
        ],
        out_shape=[
            jax.ShapeDtypeStruct((1, 1), _F32),
            jax.ShapeDtypeStruct((BATCH * TOPK, LEN * DIM), _F32),
        ],
        scratch_shapes=[
            pltpu.VMEM((DIM, DIM), _F32),
            pltpu.VMEM((1, DIM), _F32),
        ],
    )(bp_pad)

    batched_prompt = bp.reshape(BATCH, TOPK * LEN, DIM)
    return (batched_prompt, sim, acc.reshape(()), loss.reshape(()), idx)
